# Initial kernel scaffold; baseline (speedup 1.0000x reference)
import jax
import jax.numpy as jnp
from jax.experimental import pallas as pl


def kernel(x, edge_index, edge_attr, W_gcn, b_gcn, bn1_g, bn1_b, W1, b1, W2, b2, bn2_g, bn2_b):
    raise NotImplementedError("write your pallas kernel here")


# speedup vs baseline: 13.2627x; 13.2627x over previous
import functools

import jax
import jax.numpy as jnp
from jax import lax
from jax.experimental import pallas as pl
from jax.experimental.pallas import tpu as pltpu
from jax.experimental.pallas import tpu_sc as plsc

N_NODES = 10000
N_EDGES = 320000
DIM = 128
BN_EPS = 1e-5

NC, NS, L = 2, 16, 16
NW = NC * NS
CHUNK = 128
N_PAD = 10240
E_PAD = ((N_EDGES + NW * CHUNK - 1) // (NW * CHUNK)) * (NW * CHUNK)
E_PER_W = E_PAD // NW
N_CHUNKS = E_PER_W // CHUNK
ROWS_PER_TILE = N_PAD // NS

_mesh = plsc.VectorSubcoreMesh(
    core_axis_name="c", subcore_axis_name="s", num_cores=NC, num_subcores=NS)


@functools.partial(
    pl.kernel,
    out_type=jax.ShapeDtypeStruct((NC * N_PAD,), jnp.float32),
    mesh=_mesh,
    scratch_types=[
        pltpu.VMEM((CHUNK,), jnp.int32),
        pltpu.VMEM((CHUNK,), jnp.float32),
        pltpu.VMEM_SHARED((N_PAD,), jnp.float32),
    ],
)
def _deg_kernel(dst_hbm, zeros1_hbm, deg_out, idx_v, ones_v, deg_sh):
    cid = lax.axis_index("c")
    sid = lax.axis_index("s")
    wid = cid * NS + sid
    for i in range(CHUNK // L):
        ones_v[pl.ds(i * L, L)] = jnp.full((L,), 1.0, jnp.float32)
    r0 = pl.multiple_of(sid * ROWS_PER_TILE, 128)
    pltpu.sync_copy(zeros1_hbm.at[pl.ds(r0, ROWS_PER_TILE)],
                    deg_sh.at[pl.ds(r0, ROWS_PER_TILE)])
    plsc.subcore_barrier()
    base = wid * E_PER_W

    def body(c, _):
        off = base + c * CHUNK
        pltpu.sync_copy(dst_hbm.at[pl.ds(off, CHUNK)], idx_v)
        pltpu.sync_copy(ones_v, deg_sh.at[idx_v], add=True)
        return ()

    lax.fori_loop(0, N_CHUNKS, body, (), unroll=False)
    plsc.subcore_barrier()
    o0 = pl.multiple_of(cid * N_PAD + r0, 128)
    pltpu.sync_copy(deg_sh.at[pl.ds(r0, ROWS_PER_TILE)],
                    deg_out.at[pl.ds(o0, ROWS_PER_TILE)])


@functools.partial(
    pl.kernel,
    out_type=jax.ShapeDtypeStruct((NC * N_PAD, DIM), jnp.float32),
    mesh=_mesh,
    scratch_types=[
        pltpu.VMEM((CHUNK,), jnp.int32),
        pltpu.VMEM((CHUNK,), jnp.int32),
        pltpu.VMEM((CHUNK, DIM), jnp.float32),
        pltpu.VMEM_SHARED((N_PAD, DIM), jnp.float32),
        pltpu.SemaphoreType.DMA,
    ],
)
def _gather_scatter_kernel(xs_hbm, src_hbm, dst_hbm, zeros2_hbm, acc_out,
                           sidx_v, didx_v, rows_v, acc_sh, sem):
    cid = lax.axis_index("c")
    sid = lax.axis_index("s")
    wid = cid * NS + sid
    r0 = pl.multiple_of(sid * ROWS_PER_TILE, 128)
    pltpu.sync_copy(zeros2_hbm.at[pl.ds(r0, ROWS_PER_TILE)],
                    acc_sh.at[pl.ds(r0, ROWS_PER_TILE)])
    plsc.subcore_barrier()
    base = wid * E_PER_W

    def body(c, _):
        off = base + c * CHUNK
        pltpu.sync_copy(src_hbm.at[pl.ds(off, CHUNK)], sidx_v)
        pltpu.sync_copy(dst_hbm.at[pl.ds(off, CHUNK)], didx_v)
        pltpu.async_copy(xs_hbm.at[sidx_v], rows_v, sem).wait()
        pltpu.sync_copy(rows_v, acc_sh.at[didx_v], add=True)
        return ()

    lax.fori_loop(0, N_CHUNKS, body, (), unroll=False)
    plsc.subcore_barrier()
    o0 = pl.multiple_of(cid * N_PAD + r0, 128)
    pltpu.sync_copy(acc_sh.at[pl.ds(r0, ROWS_PER_TILE)],
                    acc_out.at[pl.ds(o0, ROWS_PER_TILE)])


ROW_BLK = 1000


def _scale_body(x_ref, w_ref, d0_ref, d1_ref, xs_ref):
    dinv = lax.rsqrt(d0_ref[...] + d1_ref[...] + 1.0)
    xw = jnp.dot(x_ref[...], w_ref[...], preferred_element_type=jnp.float32)
    xs_ref[...] = xw * dinv


def _scale_kernel(x, w, d0, d1):
    return pl.pallas_call(
        _scale_body,
        out_shape=jax.ShapeDtypeStruct((N_NODES, DIM), jnp.float32),
        grid=(N_NODES // ROW_BLK,),
        in_specs=[
            pl.BlockSpec((ROW_BLK, DIM), lambda i: (i, 0)),
            pl.BlockSpec((DIM, DIM), lambda i: (0, 0)),
            pl.BlockSpec((ROW_BLK, 1), lambda i: (i, 0)),
            pl.BlockSpec((ROW_BLK, 1), lambda i: (i, 0)),
        ],
        out_specs=pl.BlockSpec((ROW_BLK, DIM), lambda i: (i, 0)),
    )(x, w, d0, d1)


def _final_body(x_ref, xs_ref, a0_ref, a1_ref, d0_ref, d1_ref, bg_ref,
                g1_ref, be1_ref, w1_ref, b1_ref, w2_ref, b2_ref, g2_ref,
                be2_ref, out_ref):
    c = 1.0 / (1.0 + BN_EPS) ** 0.5
    dinv = lax.rsqrt(d0_ref[...] + d1_ref[...] + 1.0)
    acc = a0_ref[...] + a1_ref[...] + xs_ref[...]
    h = x_ref[...] + dinv * acc + bg_ref[...]
    h = g1_ref[...] * (h * c) + be1_ref[...]
    t = jnp.dot(h, w1_ref[...], preferred_element_type=jnp.float32)
    t = jnp.maximum(t + b1_ref[...], 0.0)
    ff = jnp.dot(t, w2_ref[...], preferred_element_type=jnp.float32)
    h = h + ff + b2_ref[...]
    out_ref[...] = g2_ref[...] * (h * c) + be2_ref[...]


def _final_kernel(x, xs, a0, a1, d0, d1, b_gcn, g1, be1, w1, b1, w2, b2,
                  g2, be2):
    row = lambda i: (i, 0)
    full = lambda shape: pl.BlockSpec(shape, lambda i: (0, 0))
    return pl.pallas_call(
        _final_body,
        out_shape=jax.ShapeDtypeStruct((N_NODES, DIM), jnp.float32),
        grid=(N_NODES // ROW_BLK,),
        in_specs=[
            pl.BlockSpec((ROW_BLK, DIM), row),
            pl.BlockSpec((ROW_BLK, DIM), row),
            pl.BlockSpec((ROW_BLK, DIM), row),
            pl.BlockSpec((ROW_BLK, DIM), row),
            pl.BlockSpec((ROW_BLK, 1), row),
            pl.BlockSpec((ROW_BLK, 1), row),
            full((1, DIM)),
            full((1, DIM)),
            full((1, DIM)),
            full((DIM, 2 * DIM)),
            full((1, 2 * DIM)),
            full((2 * DIM, DIM)),
            full((1, DIM)),
            full((1, DIM)),
            full((1, DIM)),
        ],
        out_specs=pl.BlockSpec((ROW_BLK, DIM), row),
    )(x, xs, a0, a1, d0, d1, b_gcn, g1, be1, w1, b1, w2, b2, g2, be2)


def kernel(x, edge_index, edge_attr, W_gcn, b_gcn, bn1_g, bn1_b, W1, b1,
           W2, b2, bn2_g, bn2_b):
    del edge_attr
    src = edge_index[0].astype(jnp.int32)
    dst = edge_index[1].astype(jnp.int32)
    npad = E_PAD - N_EDGES
    src = jnp.concatenate([src, jnp.zeros((npad,), jnp.int32)])
    dst = jnp.concatenate([dst, jnp.full((npad,), N_NODES, jnp.int32)])
    zeros1 = jnp.zeros((N_PAD,), jnp.float32)
    zeros2 = jnp.zeros((N_PAD, DIM), jnp.float32)

    deg = _deg_kernel(dst, zeros1).reshape(NC, N_PAD)
    d0 = deg[0, :N_NODES, None]
    d1 = deg[1, :N_NODES, None]
    xs = _scale_kernel(x, W_gcn, d0, d1)
    acc = _gather_scatter_kernel(xs, src, dst, zeros2).reshape(NC, N_PAD, DIM)
    a0 = acc[0, :N_NODES]
    a1 = acc[1, :N_NODES]
    return _final_kernel(
        x, xs, a0, a1, d0, d1,
        b_gcn[None, :], bn1_g[None, :], bn1_b[None, :],
        W1, b1[None, :], W2, b2[None, :], bn2_g[None, :], bn2_b[None, :])
